# Initial kernel scaffold; baseline (speedup 1.0000x reference)
import jax
import jax.numpy as jnp
from jax.experimental import pallas as pl


def kernel(feat, edge_index, W, attn_l, attn_r, bias):
    raise NotImplementedError("write your pallas kernel here")


# speedup vs baseline: 16.5256x; 16.5256x over previous
import functools

import jax
import jax.numpy as jnp
from jax import lax
from jax.experimental import pallas as pl
from jax.experimental.pallas import tpu as pltpu
from jax.experimental.pallas import tpu_sc as plsc

N_NODES = 10000
N_EDGES = 320000
IN_FEATS = 128
OUT_FEATS = 128
NUM_HEADS = 4
HD = NUM_HEADS * OUT_FEATS
NEG_SLOPE = 0.2

NC = 2
NS = 16
NW = NC * NS
EPT = N_EDGES // NW
NPAD = 10240
RPT = NPAD // NS

CH2 = 1000
CH3 = 40

_mesh = plsc.VectorSubcoreMesh(
    core_axis_name="c", subcore_axis_name="s", num_cores=NC, num_subcores=NS)


def _k1_body(feat_ref, w_ref, alr_ref, h_ref, elr_ref, mx_ref):
    h = jnp.dot(feat_ref[...], w_ref[...], preferred_element_type=jnp.float32)
    h_ref[...] = h
    elr = jnp.dot(h, alr_ref[...], preferred_element_type=jnp.float32)
    elr_ref[...] = elr
    bm = jnp.concatenate(
        [jnp.max(elr, axis=0, keepdims=True),
         jnp.zeros((1, 128 - 2 * NUM_HEADS), jnp.float32)], axis=1)

    @pl.when(pl.program_id(0) == 0)
    def _():
        mx_ref[...] = bm

    @pl.when(pl.program_id(0) != 0)
    def _():
        mx_ref[...] = jnp.maximum(mx_ref[...], bm)


def _k1(feat, W, ALR):
    blk = 1000
    grid = N_NODES // blk
    return pl.pallas_call(
        _k1_body,
        grid=(grid,),
        in_specs=[
            pl.BlockSpec((blk, IN_FEATS), lambda i: (i, 0)),
            pl.BlockSpec((IN_FEATS, HD), lambda i: (0, 0)),
            pl.BlockSpec((HD, 2 * NUM_HEADS), lambda i: (0, 0)),
        ],
        out_specs=[
            pl.BlockSpec((blk, HD), lambda i: (i, 0)),
            pl.BlockSpec((blk, 2 * NUM_HEADS), lambda i: (i, 0)),
            pl.BlockSpec((1, 128), lambda i: (0, 0)),
        ],
        out_shape=[
            jax.ShapeDtypeStruct((N_NODES, HD), jnp.float32),
            jax.ShapeDtypeStruct((N_NODES, 2 * NUM_HEADS), jnp.float32),
            jax.ShapeDtypeStruct((1, 128), jnp.float32),
        ],
    )(feat, W, ALR)


@functools.partial(
    pl.kernel,
    out_type=(
        jax.ShapeDtypeStruct((N_EDGES * NUM_HEADS,), jnp.float32),
        jax.ShapeDtypeStruct((NW, NPAD * NUM_HEADS), jnp.float32),
    ),
    mesh=_mesh,
    compiler_params=pltpu.CompilerParams(needs_layout_passes=False),
    scratch_types=[
        pltpu.VMEM((N_NODES * 2 * NUM_HEADS,), jnp.float32),
        pltpu.VMEM((128,), jnp.float32),
        pltpu.VMEM((CH2,), jnp.int32),
        pltpu.VMEM((CH2,), jnp.int32),
        pltpu.VMEM((CH2 * NUM_HEADS,), jnp.float32),
        pltpu.VMEM((NPAD * NUM_HEADS,), jnp.float32),
        pltpu.SemaphoreType.DMA,
    ],
)
def _k2(src_hbm, dst_hbm, elr_hbm, mx_hbm,
        eexp_hbm, den_hbm, elr_v, mx_v, src_v, dst_v, ee_v, den_v, sem):
    cid = lax.axis_index("c")
    sid = lax.axis_index("s")
    wid = sid * NC + cid
    pltpu.sync_copy(elr_hbm, elr_v)
    pltpu.sync_copy(mx_hbm, mx_v)

    lane = lax.iota(jnp.int32, 16)
    head = lane & 3
    eoff0 = lane >> 2
    z16 = jnp.zeros((16,), jnp.float32)

    def zer(i, c):
        den_v[pl.ds(i * 16, 16)] = z16
        return c

    lax.fori_loop(0, NPAD * NUM_HEADS // 16, zer, 0)

    mxl = plsc.load_gather(mx_v, [head])
    mxr = plsc.load_gather(mx_v, [head + NUM_HEADS])
    p = mxl + mxr
    cmax = jnp.where(p >= 0, p, NEG_SLOPE * p)
    masks = [eoff0 == i for i in range(4)]
    ebase = wid * EPT

    def chunk(k, carry):
        base = pl.multiple_of(ebase + k * CH2, 8)
        pltpu.sync_copy(src_hbm.at[pl.ds(base, CH2)], src_v)
        pltpu.sync_copy(dst_hbm.at[pl.ds(base, CH2)], dst_v)

        def grp(g, c2):
            eo = g * 4 + eoff0
            sg = plsc.load_gather(src_v, [eo])
            dg = plsc.load_gather(dst_v, [eo])
            elg = plsc.load_gather(elr_v, [sg * (2 * NUM_HEADS) + head])
            erg = plsc.load_gather(elr_v,
                                   [dg * (2 * NUM_HEADS) + NUM_HEADS + head])
            pre = elg + erg
            e = jnp.where(pre >= 0, pre, NEG_SLOPE * pre)
            ee = jnp.exp(e - cmax)
            ee_v[pl.ds(g * 16, 16)] = ee
            didx = dg * NUM_HEADS + head
            for m in masks:
                plsc.addupdate_scatter(den_v, [didx], ee, mask=m)
            return c2

        lax.fori_loop(0, CH2 // 4, grp, 0)
        pltpu.sync_copy(
            ee_v, eexp_hbm.at[pl.ds(pl.multiple_of(base * NUM_HEADS, 8), CH2 * NUM_HEADS)])
        return carry

    lax.fori_loop(0, EPT // CH2, chunk, 0)
    pltpu.sync_copy(den_v, den_hbm.at[wid])


def _k2b_body(d_ref, r_ref):
    r_ref[...] = 1.0 / jnp.sum(d_ref[...], axis=0)


def _k2b(denp):
    return pl.pallas_call(
        _k2b_body,
        out_shape=jax.ShapeDtypeStruct((NPAD * NUM_HEADS // 128, 128),
                                       jnp.float32),
    )(denp)


@functools.partial(
    pl.kernel,
    out_type=jax.ShapeDtypeStruct((N_EDGES * NUM_HEADS,), jnp.float32),
    mesh=_mesh,
    compiler_params=pltpu.CompilerParams(needs_layout_passes=False),
    scratch_types=[
        pltpu.VMEM((NPAD * NUM_HEADS,), jnp.float32),
        pltpu.VMEM((CH2,), jnp.int32),
        pltpu.VMEM((CH2 * NUM_HEADS,), jnp.float32),
        pltpu.VMEM((CH2 * NUM_HEADS,), jnp.float32),
        pltpu.SemaphoreType.DMA,
    ],
)
def _k2c(dst_hbm, eexp_hbm, rden_hbm, alpha_hbm,
         rden_v, dst_v, ee_v, al_v, sem):
    cid = lax.axis_index("c")
    sid = lax.axis_index("s")
    wid = sid * NC + cid
    pltpu.sync_copy(rden_hbm, rden_v)
    lane = lax.iota(jnp.int32, 16)
    head = lane & 3
    eoff0 = lane >> 2
    ebase = wid * EPT

    def chunk(k, carry):
        base = pl.multiple_of(ebase + k * CH2, 8)
        pltpu.sync_copy(dst_hbm.at[pl.ds(base, CH2)], dst_v)
        pltpu.sync_copy(
            eexp_hbm.at[pl.ds(pl.multiple_of(base * NUM_HEADS, 8), CH2 * NUM_HEADS)], ee_v)

        def grp(g, c2):
            eo = g * 4 + eoff0
            dg = plsc.load_gather(dst_v, [eo])
            rg = plsc.load_gather(rden_v, [dg * NUM_HEADS + head])
            al_v[pl.ds(g * 16, 16)] = ee_v[pl.ds(g * 16, 16)] * rg
            return c2

        lax.fori_loop(0, CH2 // 4, grp, 0)
        pltpu.sync_copy(
            al_v, alpha_hbm.at[pl.ds(pl.multiple_of(base * NUM_HEADS, 8), CH2 * NUM_HEADS)])
        return carry

    lax.fori_loop(0, EPT // CH2, chunk, 0)


@functools.partial(
    pl.kernel,
    out_type=jax.ShapeDtypeStruct((NC, NPAD, OUT_FEATS), jnp.float32),
    mesh=_mesh,
    compiler_params=pltpu.CompilerParams(needs_layout_passes=False),
    scratch_types=[
        pltpu.VMEM((CH3,), jnp.int32),
        pltpu.VMEM((CH3,), jnp.int32),
        pltpu.VMEM((CH3 * NUM_HEADS + 16,), jnp.float32),
        pltpu.VMEM((CH3, HD), jnp.float32),
        pltpu.VMEM((CH3, OUT_FEATS), jnp.float32),
        pltpu.VMEM_SHARED((NPAD, OUT_FEATS), jnp.float32),
        pltpu.SemaphoreType.DMA,
    ],
)
def _k3(src_hbm, dst_hbm, alpha_hbm, h_hbm, z128_hbm, acc_hbm,
        src_v, dst_v, al_v, hbuf, msg_v, acc_sp, sem):
    cid = lax.axis_index("c")
    sid = lax.axis_index("s")
    wid = sid * NC + cid
    pltpu.sync_copy(z128_hbm.at[pl.ds(sid * RPT, RPT)],
                    acc_sp.at[pl.ds(sid * RPT, RPT)])
    plsc.subcore_barrier()
    ebase = wid * EPT

    def chunk(k, carry):
        base = pl.multiple_of(ebase + k * CH3, 8)
        pltpu.sync_copy(src_hbm.at[pl.ds(base, CH3)], src_v)
        pltpu.sync_copy(dst_hbm.at[pl.ds(base, CH3)], dst_v)
        pltpu.sync_copy(
            alpha_hbm.at[pl.ds(pl.multiple_of(base * NUM_HEADS, 8), CH3 * NUM_HEADS)],
            al_v.at[pl.ds(0, CH3 * NUM_HEADS)])
        pltpu.async_copy(h_hbm.at[src_v], hbuf, sem).wait()

        def edge(e, c2):
            av = al_v[pl.ds(e * NUM_HEADS, 16)]
            a0 = av[0]
            a1 = av[1]
            a2 = av[2]
            a3 = av[3]
            for cc in range(OUT_FEATS // 16):
                off = cc * 16
                v = (a0 * hbuf[e, pl.ds(off, 16)]
                     + a1 * hbuf[e, pl.ds(128 + off, 16)]
                     + a2 * hbuf[e, pl.ds(256 + off, 16)]
                     + a3 * hbuf[e, pl.ds(384 + off, 16)])
                msg_v[e, pl.ds(off, 16)] = v
            return c2

        lax.fori_loop(0, CH3, edge, 0)
        pltpu.sync_copy(msg_v, acc_sp.at[dst_v], add=True)
        return carry

    lax.fori_loop(0, EPT // CH3, chunk, 0)
    plsc.subcore_barrier()

    @pl.when(sid == 0)
    def _():
        pltpu.sync_copy(acc_sp, acc_hbm.at[cid])


def _k4_body(a0_ref, a1_ref, b_ref, o_ref):
    bm = jnp.mean(b_ref[...], axis=0, keepdims=True)
    o_ref[...] = 0.25 * (a0_ref[...] + a1_ref[...]) + bm


def _k4(acc0, acc1, bias_hw):
    blk = 1000
    grid = N_NODES // blk
    return pl.pallas_call(
        _k4_body,
        grid=(grid,),
        in_specs=[
            pl.BlockSpec((blk, OUT_FEATS), lambda i: (i, 0)),
            pl.BlockSpec((blk, OUT_FEATS), lambda i: (i, 0)),
            pl.BlockSpec((NUM_HEADS, OUT_FEATS), lambda i: (0, 0)),
        ],
        out_specs=pl.BlockSpec((blk, OUT_FEATS), lambda i: (i, 0)),
        out_shape=jax.ShapeDtypeStruct((N_NODES, OUT_FEATS), jnp.float32),
    )(acc0, acc1, bias_hw)


def kernel(feat, edge_index, W, attn_l, attn_r, bias):
    src = edge_index[0]
    dst = edge_index[1]
    al = attn_l.reshape(NUM_HEADS, OUT_FEATS)
    ar = attn_r.reshape(NUM_HEADS, OUT_FEATS)
    eye = jnp.eye(NUM_HEADS, dtype=jnp.float32)
    alr_l = (eye[:, None, :] * al[:, :, None]).reshape(HD, NUM_HEADS)
    alr_r = (eye[:, None, :] * ar[:, :, None]).reshape(HD, NUM_HEADS)
    ALR = jnp.concatenate([alr_l, alr_r], axis=1)

    h, elr, mx = _k1(feat, W, ALR)

    eexp, denp = _k2(src, dst, elr.reshape(-1), mx.reshape(-1))

    rden = _k2b(denp.reshape(NW, NPAD * NUM_HEADS // 128, 128))

    alpha = _k2c(dst, eexp, rden.reshape(-1))

    z128 = jnp.zeros((NPAD, OUT_FEATS), jnp.float32)
    accp = _k3(src, dst, alpha, h, z128)

    h_out = _k4(accp[0], accp[1], bias.reshape(NUM_HEADS, OUT_FEATS))
    return h_out, alpha.reshape(N_EDGES, NUM_HEADS, 1)


# speedup vs baseline: 20.3647x; 1.2323x over previous
import functools

import jax
import jax.numpy as jnp
from jax import lax
from jax.experimental import pallas as pl
from jax.experimental.pallas import tpu as pltpu
from jax.experimental.pallas import tpu_sc as plsc

N_NODES = 10000
N_EDGES = 320000
IN_FEATS = 128
OUT_FEATS = 128
NUM_HEADS = 4
HD = NUM_HEADS * OUT_FEATS
NEG_SLOPE = 0.2

NC = 2
NS = 16
NW = NC * NS
EPT = N_EDGES // NW
NPAD = 10240
RPT = NPAD // NS

CH2 = 1000
CH3 = 40

_mesh = plsc.VectorSubcoreMesh(
    core_axis_name="c", subcore_axis_name="s", num_cores=NC, num_subcores=NS)


def _k1_body(feat_ref, w_ref, alr_ref, h_ref, elr_ref, mx_ref):
    h = jnp.dot(feat_ref[...], w_ref[...], preferred_element_type=jnp.float32)
    h_ref[...] = h
    elr = jnp.dot(h, alr_ref[...], preferred_element_type=jnp.float32)
    elr_ref[...] = elr
    bm = jnp.concatenate(
        [jnp.max(elr, axis=0, keepdims=True),
         jnp.zeros((1, 128 - 2 * NUM_HEADS), jnp.float32)], axis=1)

    @pl.when(pl.program_id(0) == 0)
    def _():
        mx_ref[...] = bm

    @pl.when(pl.program_id(0) != 0)
    def _():
        mx_ref[...] = jnp.maximum(mx_ref[...], bm)


def _k1(feat, W, ALR):
    blk = 1000
    grid = N_NODES // blk
    return pl.pallas_call(
        _k1_body,
        grid=(grid,),
        in_specs=[
            pl.BlockSpec((blk, IN_FEATS), lambda i: (i, 0)),
            pl.BlockSpec((IN_FEATS, HD), lambda i: (0, 0)),
            pl.BlockSpec((HD, 2 * NUM_HEADS), lambda i: (0, 0)),
        ],
        out_specs=[
            pl.BlockSpec((blk, HD), lambda i: (i, 0)),
            pl.BlockSpec((blk, 2 * NUM_HEADS), lambda i: (i, 0)),
            pl.BlockSpec((1, 128), lambda i: (0, 0)),
        ],
        out_shape=[
            jax.ShapeDtypeStruct((N_NODES, HD), jnp.float32),
            jax.ShapeDtypeStruct((N_NODES, 2 * NUM_HEADS), jnp.float32),
            jax.ShapeDtypeStruct((1, 128), jnp.float32),
        ],
    )(feat, W, ALR)


@functools.partial(
    pl.kernel,
    out_type=(
        jax.ShapeDtypeStruct((N_EDGES * NUM_HEADS,), jnp.float32),
        jax.ShapeDtypeStruct((NW, NPAD * NUM_HEADS), jnp.float32),
    ),
    mesh=_mesh,
    compiler_params=pltpu.CompilerParams(needs_layout_passes=False),
    scratch_types=[
        pltpu.VMEM((N_NODES * 2 * NUM_HEADS,), jnp.float32),
        pltpu.VMEM((128,), jnp.float32),
        pltpu.VMEM((CH2,), jnp.int32),
        pltpu.VMEM((CH2,), jnp.int32),
        pltpu.VMEM((CH2 * NUM_HEADS,), jnp.float32),
        pltpu.VMEM((NPAD * NUM_HEADS,), jnp.float32),
        pltpu.SemaphoreType.DMA,
    ],
)
def _k2(src_hbm, dst_hbm, elr_hbm, mx_hbm,
        eexp_hbm, den_hbm, elr_v, mx_v, src_v, dst_v, ee_v, den_v, sem):
    cid = lax.axis_index("c")
    sid = lax.axis_index("s")
    wid = sid * NC + cid
    pltpu.sync_copy(elr_hbm, elr_v)
    pltpu.sync_copy(mx_hbm, mx_v)

    lane = lax.iota(jnp.int32, 16)
    head = lane & 3
    eoff0 = lane >> 2
    z16 = jnp.zeros((16,), jnp.float32)

    def zer(i, c):
        den_v[pl.ds(i * 16, 16)] = z16
        return c

    lax.fori_loop(0, NPAD * NUM_HEADS // 16, zer, 0)

    mxl = plsc.load_gather(mx_v, [head])
    mxr = plsc.load_gather(mx_v, [head + NUM_HEADS])
    p = mxl + mxr
    cmax = jnp.where(p >= 0, p, NEG_SLOPE * p)
    masks = [eoff0 == i for i in range(4)]
    ebase = wid * EPT

    def chunk(k, carry):
        base = pl.multiple_of(ebase + k * CH2, 8)
        pltpu.sync_copy(src_hbm.at[pl.ds(base, CH2)], src_v)
        pltpu.sync_copy(dst_hbm.at[pl.ds(base, CH2)], dst_v)

        def grp(g, c2):
            eo = g * 4 + eoff0
            sg = plsc.load_gather(src_v, [eo])
            dg = plsc.load_gather(dst_v, [eo])
            elg = plsc.load_gather(elr_v, [sg * (2 * NUM_HEADS) + head])
            erg = plsc.load_gather(elr_v,
                                   [dg * (2 * NUM_HEADS) + NUM_HEADS + head])
            pre = elg + erg
            e = jnp.where(pre >= 0, pre, NEG_SLOPE * pre)
            ee = jnp.exp(e - cmax)
            ee_v[pl.ds(g * 16, 16)] = ee
            didx = dg * NUM_HEADS + head
            for m in masks:
                plsc.addupdate_scatter(den_v, [didx], ee, mask=m)
            return c2

        lax.fori_loop(0, CH2 // 4, grp, 0)
        pltpu.sync_copy(
            ee_v, eexp_hbm.at[pl.ds(pl.multiple_of(base * NUM_HEADS, 8), CH2 * NUM_HEADS)])
        return carry

    lax.fori_loop(0, EPT // CH2, chunk, 0)
    pltpu.sync_copy(den_v, den_hbm.at[wid])


def _k2b_body(d_ref, r_ref):
    r_ref[...] = 1.0 / jnp.sum(d_ref[...], axis=0)


def _k2b(denp):
    return pl.pallas_call(
        _k2b_body,
        out_shape=jax.ShapeDtypeStruct((NPAD * NUM_HEADS // 128, 128),
                                       jnp.float32),
    )(denp)


@functools.partial(
    pl.kernel,
    out_type=jax.ShapeDtypeStruct((N_EDGES * NUM_HEADS,), jnp.float32),
    mesh=_mesh,
    compiler_params=pltpu.CompilerParams(needs_layout_passes=False),
    scratch_types=[
        pltpu.VMEM((NPAD * NUM_HEADS,), jnp.float32),
        pltpu.VMEM((CH2,), jnp.int32),
        pltpu.VMEM((CH2 * NUM_HEADS,), jnp.float32),
        pltpu.VMEM((CH2 * NUM_HEADS,), jnp.float32),
        pltpu.SemaphoreType.DMA,
    ],
)
def _k2c(dst_hbm, eexp_hbm, rden_hbm, alpha_hbm,
         rden_v, dst_v, ee_v, al_v, sem):
    cid = lax.axis_index("c")
    sid = lax.axis_index("s")
    wid = sid * NC + cid
    pltpu.sync_copy(rden_hbm, rden_v)
    lane = lax.iota(jnp.int32, 16)
    head = lane & 3
    eoff0 = lane >> 2
    ebase = wid * EPT

    def chunk(k, carry):
        base = pl.multiple_of(ebase + k * CH2, 8)
        pltpu.sync_copy(dst_hbm.at[pl.ds(base, CH2)], dst_v)
        pltpu.sync_copy(
            eexp_hbm.at[pl.ds(pl.multiple_of(base * NUM_HEADS, 8), CH2 * NUM_HEADS)], ee_v)

        def grp(g, c2):
            eo = g * 4 + eoff0
            dg = plsc.load_gather(dst_v, [eo])
            rg = plsc.load_gather(rden_v, [dg * NUM_HEADS + head])
            al_v[pl.ds(g * 16, 16)] = ee_v[pl.ds(g * 16, 16)] * rg
            return c2

        lax.fori_loop(0, CH2 // 4, grp, 0)
        pltpu.sync_copy(
            al_v, alpha_hbm.at[pl.ds(pl.multiple_of(base * NUM_HEADS, 8), CH2 * NUM_HEADS)])
        return carry

    lax.fori_loop(0, EPT // CH2, chunk, 0)


BE = 80
NCH = BE // CH3
NBLK = EPT // BE


@functools.partial(
    pl.kernel,
    out_type=jax.ShapeDtypeStruct((NC, NPAD, OUT_FEATS), jnp.float32),
    mesh=_mesh,
    compiler_params=pltpu.CompilerParams(needs_layout_passes=False),
    scratch_types=[
        pltpu.VMEM((BE,), jnp.int32),
        pltpu.VMEM((NCH, CH3), jnp.int32),
        pltpu.VMEM((BE * NUM_HEADS + 16,), jnp.float32),
        pltpu.VMEM((CH3, HD), jnp.float32),
        pltpu.VMEM((CH3, HD), jnp.float32),
        pltpu.VMEM((CH3, OUT_FEATS), jnp.float32),
        pltpu.VMEM_SHARED((NPAD, OUT_FEATS), jnp.float32),
        pltpu.SemaphoreType.DMA,
        pltpu.SemaphoreType.DMA,
    ],
)
def _k3(src_hbm, dst2_hbm, alpha_hbm, h_hbm, z128_hbm, acc_hbm,
        src_v, dst_v, al_v, hbuf0, hbuf1, msg_v, acc_sp, sem0, sem1):
    cid = lax.axis_index("c")
    sid = lax.axis_index("s")
    wid = sid * NC + cid
    pltpu.sync_copy(z128_hbm.at[pl.ds(sid * RPT, RPT)],
                    acc_sp.at[pl.ds(sid * RPT, RPT)])
    plsc.subcore_barrier()
    ebase = wid * EPT

    def start(jj, buf, sem):
        idx = src_v.at[pl.ds(pl.multiple_of(jj * CH3, 8), CH3)]
        pltpu.async_copy(h_hbm.at[idx], buf, sem)

    def wait(buf, sem):
        idx = src_v.at[pl.ds(0, CH3)]
        pltpu.make_async_copy(h_hbm.at[idx], buf, sem).wait()

    def do_chunk(jj, buf):
        def edge(eC, c2):
            eL = jj * CH3 + eC
            av = al_v[pl.ds(eL * NUM_HEADS, 16)]
            a0 = av[0]
            a1 = av[1]
            a2 = av[2]
            a3 = av[3]
            for cc in range(OUT_FEATS // 16):
                off = cc * 16
                v = (a0 * buf[eC, pl.ds(off, 16)]
                     + a1 * buf[eC, pl.ds(128 + off, 16)]
                     + a2 * buf[eC, pl.ds(256 + off, 16)]
                     + a3 * buf[eC, pl.ds(384 + off, 16)])
                msg_v[eC, pl.ds(off, 16)] = v
            return c2

        lax.fori_loop(0, CH3, edge, 0)
        pltpu.sync_copy(msg_v, acc_sp.at[dst_v.at[jj]], add=True)

    def block(b, carry):
        bb = pl.multiple_of(ebase + b * BE, 8)
        pltpu.sync_copy(src_hbm.at[pl.ds(bb, BE)], src_v)
        pltpu.sync_copy(dst2_hbm.at[wid * NBLK + b], dst_v)
        pltpu.sync_copy(
            alpha_hbm.at[pl.ds(pl.multiple_of(bb * NUM_HEADS, 8),
                               BE * NUM_HEADS)],
            al_v.at[pl.ds(0, BE * NUM_HEADS)])
        start(0, hbuf0, sem0)

        def pair(p, c2):
            j0 = p * 2
            wait(hbuf0, sem0)
            start(j0 + 1, hbuf1, sem1)
            do_chunk(j0, hbuf0)
            wait(hbuf1, sem1)

            @pl.when(p < NCH // 2 - 1)
            def _():
                start(j0 + 2, hbuf0, sem0)

            do_chunk(j0 + 1, hbuf1)
            return c2

        lax.fori_loop(0, NCH // 2, pair, 0)
        return carry

    lax.fori_loop(0, NBLK, block, 0)
    plsc.subcore_barrier()

    @pl.when(sid == 0)
    def _():
        pltpu.sync_copy(acc_sp, acc_hbm.at[cid])


def _k4_body(a0_ref, a1_ref, b_ref, o_ref):
    bm = jnp.mean(b_ref[...], axis=0, keepdims=True)
    o_ref[...] = 0.25 * (a0_ref[...] + a1_ref[...]) + bm


def _k4(acc0, acc1, bias_hw):
    blk = 1000
    grid = N_NODES // blk
    return pl.pallas_call(
        _k4_body,
        grid=(grid,),
        in_specs=[
            pl.BlockSpec((blk, OUT_FEATS), lambda i: (i, 0)),
            pl.BlockSpec((blk, OUT_FEATS), lambda i: (i, 0)),
            pl.BlockSpec((NUM_HEADS, OUT_FEATS), lambda i: (0, 0)),
        ],
        out_specs=pl.BlockSpec((blk, OUT_FEATS), lambda i: (i, 0)),
        out_shape=jax.ShapeDtypeStruct((N_NODES, OUT_FEATS), jnp.float32),
    )(acc0, acc1, bias_hw)


def kernel(feat, edge_index, W, attn_l, attn_r, bias):
    src = edge_index[0]
    dst = edge_index[1]
    al = attn_l.reshape(NUM_HEADS, OUT_FEATS)
    ar = attn_r.reshape(NUM_HEADS, OUT_FEATS)
    eye = jnp.eye(NUM_HEADS, dtype=jnp.float32)
    alr_l = (eye[:, None, :] * al[:, :, None]).reshape(HD, NUM_HEADS)
    alr_r = (eye[:, None, :] * ar[:, :, None]).reshape(HD, NUM_HEADS)
    ALR = jnp.concatenate([alr_l, alr_r], axis=1)

    h, elr, mx = _k1(feat, W, ALR)

    eexp, denp = _k2(src, dst, elr.reshape(-1), mx.reshape(-1))

    rden = _k2b(denp.reshape(NW, NPAD * NUM_HEADS // 128, 128))

    alpha = _k2c(dst, eexp, rden.reshape(-1))

    z128 = jnp.zeros((NPAD, OUT_FEATS), jnp.float32)
    accp = _k3(src, dst.reshape(NW * NBLK, NCH, CH3), alpha, h, z128)

    h_out = _k4(accp[0], accp[1], bias.reshape(NUM_HEADS, OUT_FEATS))
    return h_out, alpha.reshape(N_EDGES, NUM_HEADS, 1)
